# baseline (speedup 1.0000x reference)
import jax
import jax.numpy as jnp
from jax.experimental import pallas as pl

B, C, H, W = 64, 96, 32, 32
D2 = C // 2


def _body(x_ref, row_ref, col_ref, out_ref):
    top = jnp.broadcast_to(col_ref[...][None, :, :], (H, W, D2))
    bot = jnp.broadcast_to(row_ref[...][:, None, :], (H, W, D2))
    pos = jnp.concatenate([top, bot], axis=-1)
    out_ref[...] = x_ref[...] + pos[None]


@jax.jit
def kernel(x, row_table, col_table):
    xt = x.transpose(0, 2, 3, 1)
    row_e = row_table[:H]
    col_e = col_table[:W]

    bblk = 8
    out = pl.pallas_call(
        _body,
        grid=(B // bblk,),
        in_specs=[
            pl.BlockSpec((bblk, H, W, C), lambda i: (i, 0, 0, 0)),
            pl.BlockSpec((H, D2), lambda i: (0, 0)),
            pl.BlockSpec((W, D2), lambda i: (0, 0)),
        ],
        out_specs=pl.BlockSpec((bblk, H, W, C), lambda i: (i, 0, 0, 0)),
        out_shape=jax.ShapeDtypeStruct((B, H, W, C), jnp.float32),
    )(xt, row_e, col_e)
    return out.transpose(0, 3, 1, 2)


# speedup vs baseline: 1.1332x; 1.1332x over previous
import jax
import jax.numpy as jnp
from jax.experimental import pallas as pl

B, C, H, W = 64, 96, 32, 32
D2 = C // 2


def _body(x_ref, row_ref, col_ref, out_ref):
    col_e = jnp.transpose(col_ref[:, 0:W], (1, 0))
    row_e = jnp.transpose(row_ref[:, 0:H], (1, 0))
    top = jnp.broadcast_to(col_e[None, :, :], (H, W, D2))
    bot = jnp.broadcast_to(row_e[:, None, :], (H, W, D2))
    pos = jnp.concatenate([top, bot], axis=-1)
    out_ref[...] = x_ref[...] + pos[None]


@jax.jit
def kernel(x, row_table, col_table):
    xt = x.transpose(0, 2, 3, 1)
    rt = row_table.T
    ct = col_table.T

    bblk = 8
    out = pl.pallas_call(
        _body,
        grid=(B // bblk,),
        in_specs=[
            pl.BlockSpec((bblk, H, W, C), lambda i: (i, 0, 0, 0)),
            pl.BlockSpec(rt.shape, lambda i: (0, 0)),
            pl.BlockSpec(ct.shape, lambda i: (0, 0)),
        ],
        out_specs=pl.BlockSpec((bblk, H, W, C), lambda i: (i, 0, 0, 0)),
        out_shape=jax.ShapeDtypeStruct((B, H, W, C), jnp.float32),
    )(xt, rt, ct)
    return out.transpose(0, 3, 1, 2)


# speedup vs baseline: 1.1415x; 1.0074x over previous
import jax
import jax.numpy as jnp
from jax.experimental import pallas as pl

B, C, H, W = 64, 96, 32, 32
D2 = C // 2


def _body(x_ref, row_ref, col_ref, out_ref):
    col_e = jnp.transpose(col_ref[:, 0:W], (1, 0))
    row_e = jnp.transpose(row_ref[:, 0:H], (1, 0))
    top = jnp.broadcast_to(col_e[None, :, :], (H, W, D2))
    bot = jnp.broadcast_to(row_e[:, None, :], (H, W, D2))
    pos = jnp.concatenate([top, bot], axis=-1)
    out_ref[...] = x_ref[...] + pos[None]


@jax.jit
def kernel(x, row_table, col_table):
    xt = x.transpose(0, 2, 3, 1)
    rt = row_table.T
    ct = col_table.T

    bblk = 16
    out = pl.pallas_call(
        _body,
        grid=(B // bblk,),
        in_specs=[
            pl.BlockSpec((bblk, H, W, C), lambda i: (i, 0, 0, 0)),
            pl.BlockSpec(rt.shape, lambda i: (0, 0)),
            pl.BlockSpec(ct.shape, lambda i: (0, 0)),
        ],
        out_specs=pl.BlockSpec((bblk, H, W, C), lambda i: (i, 0, 0, 0)),
        out_shape=jax.ShapeDtypeStruct((B, H, W, C), jnp.float32),
    )(xt, rt, ct)
    return out.transpose(0, 3, 1, 2)
